# initial kernel scaffold (unmeasured)
import jax
import jax.numpy as jnp
from jax import lax
from jax.experimental import pallas as pl
from jax.experimental.pallas import tpu as pltpu

N_DEV = 4


def kernel(x, w_mat):
    m_glob, k_loc = x.shape
    k_glob, n = w_mat.shape
    m_per = m_glob // N_DEV

    def body(x_ref, w_ref, out_ref, comm_ref, amax_ref,
             data_send_sems, data_recv_sems,
             amax_send_sems, amax_recv_sems):
        my = lax.axis_index("i")

        barrier_sem = pltpu.get_barrier_semaphore()
        for off in range(1, N_DEV):
            peer = (my + off) % N_DEV
            pl.semaphore_signal(
                barrier_sem, inc=1,
                device_id=(peer,), device_id_type=pl.DeviceIdType.MESH,
            )
        pl.semaphore_wait(barrier_sem, N_DEV - 1)

        sends = []
        for off in range(1, N_DEV):
            dst = (my + off) % N_DEV
            rdma = pltpu.make_async_remote_copy(
                src_ref=x_ref.at[pl.ds(dst * m_per, m_per), :],
                dst_ref=comm_ref.at[my],
                send_sem=data_send_sems.at[off - 1],
                recv_sem=data_recv_sems.at[my],
                device_id=(dst,),
                device_id_type=pl.DeviceIdType.MESH,
            )
            rdma.start()
            sends.append(rdma)

        comm_ref[my] = x_ref[pl.ds(my * m_per, m_per), :]

        out_ref[...] = jnp.zeros((m_per, n), jnp.float32)
        for d in range(N_DEV):
            recv = pltpu.make_async_remote_copy(
                src_ref=comm_ref.at[d],
                dst_ref=comm_ref.at[d],
                send_sem=data_send_sems.at[0],
                recv_sem=data_recv_sems.at[d],
                device_id=(my,),
                device_id_type=pl.DeviceIdType.MESH,
            )

            @pl.when(my != d)
            def _():
                recv.wait_recv()

            out_ref[...] += jnp.dot(
                comm_ref[d],
                w_ref[d * k_loc:(d + 1) * k_loc, :],
                preferred_element_type=jnp.float32,
            )

        for rdma in sends:
            rdma.wait_send()

        local_amax = jnp.max(jnp.abs(out_ref[...]))
        amax_ref[my] = jnp.full((8, 128), local_amax, jnp.float32)
        amax_sends = []
        for off in range(1, N_DEV):
            dst = (my + off) % N_DEV
            rdma = pltpu.make_async_remote_copy(
                src_ref=amax_ref.at[my],
                dst_ref=amax_ref.at[my],
                send_sem=amax_send_sems.at[off - 1],
                recv_sem=amax_recv_sems.at[my],
                device_id=(dst,),
                device_id_type=pl.DeviceIdType.MESH,
            )
            rdma.start()
            amax_sends.append(rdma)
        for d in range(N_DEV):
            recv = pltpu.make_async_remote_copy(
                src_ref=amax_ref.at[d],
                dst_ref=amax_ref.at[d],
                send_sem=amax_send_sems.at[0],
                recv_sem=amax_recv_sems.at[d],
                device_id=(my,),
                device_id_type=pl.DeviceIdType.MESH,
            )

            @pl.when(my != d)
            def _():
                recv.wait_recv()

        for rdma in amax_sends:
            rdma.wait_send()

        g_amax = jnp.max(amax_ref[...])
        scale = g_amax / 448.0
        q = jnp.clip(out_ref[...] / scale, -448.0, 448.0)
        q = q.astype(jnp.float8_e4m3fn).astype(jnp.float32)
        out_ref[...] = q * scale

    return pl.pallas_call(
        body,
        out_shape=jax.ShapeDtypeStruct((m_per, n), jnp.float32),
        in_specs=[
            pl.BlockSpec(memory_space=pltpu.VMEM),
            pl.BlockSpec(memory_space=pltpu.VMEM),
        ],
        out_specs=pl.BlockSpec(memory_space=pltpu.VMEM),
        scratch_shapes=[
            pltpu.VMEM((N_DEV, m_per, k_loc), jnp.float32),
            pltpu.VMEM((N_DEV, 8, 128), jnp.float32),
            pltpu.SemaphoreType.DMA((N_DEV - 1,)),
            pltpu.SemaphoreType.DMA((N_DEV,)),
            pltpu.SemaphoreType.DMA((N_DEV - 1,)),
            pltpu.SemaphoreType.DMA((N_DEV,)),
        ],
        compiler_params=pltpu.CompilerParams(collective_id=0),
    )(x, w_mat)


# baseline (device time: 129244 ns/iter reference)
import jax
import jax.numpy as jnp
from jax import lax
from jax.experimental import pallas as pl
from jax.experimental.pallas import tpu as pltpu

N_DEV = 4


def kernel(x, w_mat):
    m_glob, k_loc = x.shape
    k_glob, n = w_mat.shape
    m_per = m_glob // N_DEV

    def body(x_ref, w_ref, out_ref, comm_ref, wbuf_ref, amax_ref,
             data_send_sems, data_recv_sems,
             amax_send_sems, amax_recv_sems,
             x_copy_sem, w_copy_sems):
        my = lax.axis_index("i")

        barrier_sem = pltpu.get_barrier_semaphore()
        for off in range(1, N_DEV):
            peer = (my + off) % N_DEV
            pl.semaphore_signal(
                barrier_sem, inc=1,
                device_id=(peer,), device_id_type=pl.DeviceIdType.MESH,
            )
        pl.semaphore_wait(barrier_sem, N_DEV - 1)

        sends = []
        for off in range(1, N_DEV):
            dst = (my + off) % N_DEV
            rdma = pltpu.make_async_remote_copy(
                src_ref=x_ref.at[pl.ds(dst * m_per, m_per), :],
                dst_ref=comm_ref.at[my],
                send_sem=data_send_sems.at[off - 1],
                recv_sem=data_recv_sems.at[my],
                device_id=(dst,),
                device_id_type=pl.DeviceIdType.MESH,
            )
            rdma.start()
            sends.append(rdma)

        x_copy = pltpu.make_async_copy(
            x_ref.at[pl.ds(my * m_per, m_per), :],
            comm_ref.at[my],
            x_copy_sem,
        )
        x_copy.start()

        w_copies = []
        for d in range(N_DEV):
            w_copies.append(pltpu.make_async_copy(
                w_ref.at[pl.ds(d * k_loc, k_loc), :],
                wbuf_ref.at[d % 2],
                w_copy_sems.at[d % 2],
            ))
        w_copies[0].start()
        x_copy.wait()

        out_ref[...] = jnp.zeros((m_per, n), jnp.float32)
        for d in range(N_DEV):
            if d + 1 < N_DEV:
                w_copies[d + 1].start()
            w_copies[d].wait()

            recv = pltpu.make_async_remote_copy(
                src_ref=comm_ref.at[d],
                dst_ref=comm_ref.at[d],
                send_sem=data_send_sems.at[0],
                recv_sem=data_recv_sems.at[d],
                device_id=(my,),
                device_id_type=pl.DeviceIdType.MESH,
            )

            @pl.when(my != d)
            def _():
                recv.wait_recv()

            out_ref[...] += jnp.dot(
                comm_ref[d],
                wbuf_ref[d % 2],
                preferred_element_type=jnp.float32,
            )

        for rdma in sends:
            rdma.wait_send()

        local_amax = jnp.max(jnp.abs(out_ref[...]))
        amax_ref[my] = jnp.full((8, 128), local_amax, jnp.float32)
        amax_sends = []
        for off in range(1, N_DEV):
            dst = (my + off) % N_DEV
            rdma = pltpu.make_async_remote_copy(
                src_ref=amax_ref.at[my],
                dst_ref=amax_ref.at[my],
                send_sem=amax_send_sems.at[off - 1],
                recv_sem=amax_recv_sems.at[my],
                device_id=(dst,),
                device_id_type=pl.DeviceIdType.MESH,
            )
            rdma.start()
            amax_sends.append(rdma)
        for d in range(N_DEV):
            recv = pltpu.make_async_remote_copy(
                src_ref=amax_ref.at[d],
                dst_ref=amax_ref.at[d],
                send_sem=amax_send_sems.at[0],
                recv_sem=amax_recv_sems.at[d],
                device_id=(my,),
                device_id_type=pl.DeviceIdType.MESH,
            )

            @pl.when(my != d)
            def _():
                recv.wait_recv()

        for rdma in amax_sends:
            rdma.wait_send()

        g_amax = jnp.max(amax_ref[...])
        scale = g_amax / 448.0
        q = jnp.clip(out_ref[...] / scale, -448.0, 448.0)
        q = q.astype(jnp.float8_e4m3fn).astype(jnp.float32)
        out_ref[...] = q * scale

    return pl.pallas_call(
        body,
        out_shape=jax.ShapeDtypeStruct((m_per, n), jnp.float32),
        in_specs=[
            pl.BlockSpec(memory_space=pl.ANY),
            pl.BlockSpec(memory_space=pl.ANY),
        ],
        out_specs=pl.BlockSpec(memory_space=pltpu.VMEM),
        scratch_shapes=[
            pltpu.VMEM((N_DEV, m_per, k_loc), jnp.float32),
            pltpu.VMEM((2, k_loc, n), jnp.float32),
            pltpu.VMEM((N_DEV, 8, 128), jnp.float32),
            pltpu.SemaphoreType.DMA((N_DEV - 1,)),
            pltpu.SemaphoreType.DMA((N_DEV,)),
            pltpu.SemaphoreType.DMA((N_DEV - 1,)),
            pltpu.SemaphoreType.DMA((N_DEV,)),
            pltpu.SemaphoreType.DMA,
            pltpu.SemaphoreType.DMA((2,)),
        ],
        compiler_params=pltpu.CompilerParams(
            collective_id=0,
            vmem_limit_bytes=56 * 1024 * 1024,
        ),
    )(x, w_mat)


# device time: 114043 ns/iter; 1.1333x vs baseline; 1.1333x over previous
import jax
import jax.numpy as jnp
from jax import lax
from jax.experimental import pallas as pl
from jax.experimental.pallas import tpu as pltpu

N_DEV = 4
NCHUNK = 4
REMOTE_OFFS = (1, 3, 2)


def kernel(x, w_mat):
    m_glob, k_loc = x.shape
    k_glob, n = w_mat.shape
    m_per = m_glob // N_DEV
    m_chunk = m_per // NCHUNK

    def body(x_ref, w_ref, out_ref, comm_ref, wbuf_ref, amax_ref,
             data_send_sems, data_recv_sems,
             amax_send_sems, amax_recv_sems,
             x_copy_sem, w_copy_sems):
        my = lax.axis_index("i")

        barrier_sem = pltpu.get_barrier_semaphore()
        for off in range(1, N_DEV):
            peer = (my + off) % N_DEV
            pl.semaphore_signal(
                barrier_sem, inc=1,
                device_id=(peer,), device_id_type=pl.DeviceIdType.MESH,
            )
        pl.semaphore_wait(barrier_sem, N_DEV - 1)

        sends = []
        for c in range(NCHUNK):
            for i, off in enumerate(REMOTE_OFFS):
                dst = (my + off) % N_DEV
                rdma = pltpu.make_async_remote_copy(
                    src_ref=x_ref.at[pl.ds(dst * m_per + c * m_chunk,
                                           m_chunk), :],
                    dst_ref=comm_ref.at[my, pl.ds(c * m_chunk, m_chunk)],
                    send_sem=data_send_sems.at[i, c],
                    recv_sem=data_recv_sems.at[my, c],
                    device_id=(dst,),
                    device_id_type=pl.DeviceIdType.MESH,
                )
                rdma.start()
                sends.append(rdma)

        x_copy = pltpu.make_async_copy(
            x_ref.at[pl.ds(my * m_per, m_per), :],
            comm_ref.at[my],
            x_copy_sem,
        )
        x_copy.start()

        w_order = [my] + [(my + off) % N_DEV for off in REMOTE_OFFS]
        w_copies = []
        for i, d in enumerate(w_order):
            w_copies.append(pltpu.make_async_copy(
                w_ref.at[pl.ds(d * k_loc, k_loc), :],
                wbuf_ref.at[i % 2],
                w_copy_sems.at[i % 2],
            ))
        w_copies[0].start()
        w_copies[1].start()

        x_copy.wait()
        w_copies[0].wait()
        out_ref[...] = jnp.dot(
            comm_ref[my], wbuf_ref[0],
            preferred_element_type=jnp.float32,
        )

        for i, off in enumerate(REMOTE_OFFS):
            d = (my + off) % N_DEV
            slot = (i + 1) % 2
            if i + 2 < len(w_order):
                w_copies[i + 2].start()
            w_copies[i + 1].wait()
            for c in range(NCHUNK):
                recv = pltpu.make_async_remote_copy(
                    src_ref=comm_ref.at[d, pl.ds(c * m_chunk, m_chunk)],
                    dst_ref=comm_ref.at[d, pl.ds(c * m_chunk, m_chunk)],
                    send_sem=data_send_sems.at[i, c],
                    recv_sem=data_recv_sems.at[d, c],
                    device_id=(my,),
                    device_id_type=pl.DeviceIdType.MESH,
                )
                recv.wait_recv()
                out_ref[pl.ds(c * m_chunk, m_chunk), :] += jnp.dot(
                    comm_ref[d, pl.ds(c * m_chunk, m_chunk), :],
                    wbuf_ref[slot],
                    preferred_element_type=jnp.float32,
                )

        for rdma in sends:
            rdma.wait_send()

        local_amax = jnp.max(jnp.abs(out_ref[...]))
        amax_ref[my] = jnp.full((8, 128), local_amax, jnp.float32)
        amax_sends = []
        for off in range(1, N_DEV):
            dst = (my + off) % N_DEV
            rdma = pltpu.make_async_remote_copy(
                src_ref=amax_ref.at[my],
                dst_ref=amax_ref.at[my],
                send_sem=amax_send_sems.at[off - 1],
                recv_sem=amax_recv_sems.at[my],
                device_id=(dst,),
                device_id_type=pl.DeviceIdType.MESH,
            )
            rdma.start()
            amax_sends.append(rdma)
        for d in range(N_DEV):
            recv = pltpu.make_async_remote_copy(
                src_ref=amax_ref.at[d],
                dst_ref=amax_ref.at[d],
                send_sem=amax_send_sems.at[0],
                recv_sem=amax_recv_sems.at[d],
                device_id=(my,),
                device_id_type=pl.DeviceIdType.MESH,
            )

            @pl.when(my != d)
            def _():
                recv.wait_recv()

        for rdma in amax_sends:
            rdma.wait_send()

        g_amax = jnp.max(amax_ref[...])
        scale = g_amax / 448.0
        q = jnp.clip(out_ref[...] / scale, -448.0, 448.0)
        q = q.astype(jnp.float8_e4m3fn).astype(jnp.float32)
        out_ref[...] = q * scale

    return pl.pallas_call(
        body,
        out_shape=jax.ShapeDtypeStruct((m_per, n), jnp.float32),
        in_specs=[
            pl.BlockSpec(memory_space=pl.ANY),
            pl.BlockSpec(memory_space=pl.ANY),
        ],
        out_specs=pl.BlockSpec(memory_space=pltpu.VMEM),
        scratch_shapes=[
            pltpu.VMEM((N_DEV, m_per, k_loc), jnp.float32),
            pltpu.VMEM((2, k_loc, n), jnp.float32),
            pltpu.VMEM((N_DEV, 8, 128), jnp.float32),
            pltpu.SemaphoreType.DMA((N_DEV - 1, NCHUNK)),
            pltpu.SemaphoreType.DMA((N_DEV, NCHUNK)),
            pltpu.SemaphoreType.DMA((N_DEV - 1,)),
            pltpu.SemaphoreType.DMA((N_DEV,)),
            pltpu.SemaphoreType.DMA,
            pltpu.SemaphoreType.DMA((2,)),
        ],
        compiler_params=pltpu.CompilerParams(
            collective_id=0,
            vmem_limit_bytes=56 * 1024 * 1024,
        ),
    )(x, w_mat)


# device time: 81982 ns/iter; 1.5765x vs baseline; 1.3911x over previous
import jax
import jax.numpy as jnp
from jax import lax
from jax.experimental import pallas as pl
from jax.experimental.pallas import tpu as pltpu

N_DEV = 4
NCHUNK = 4
SEND_OFFS = (2, 1, 3)
CONSUME_OFFS = (1, 3, 2)


def kernel(x, w_mat):
    m_glob, k_loc = x.shape
    k_glob, n = w_mat.shape
    m_per = m_glob // N_DEV
    m_chunk = m_per // NCHUNK

    def body(x_ref, w_ref, out_ref, comm_ref, xsend_ref, stage_ref,
             wbuf_ref, scale_send_ref, scale_ref, amax_ref,
             data_send_sems, data_recv_sems,
             scale_send_sems, scale_recv_sems,
             amax_send_sems, amax_recv_sems,
             stage_sems, w_copy_sems):
        my = lax.axis_index("i")

        barrier_sem = pltpu.get_barrier_semaphore()
        for off in range(1, N_DEV):
            peer = (my + off) % N_DEV
            pl.semaphore_signal(
                barrier_sem, inc=1,
                device_id=(peer,), device_id_type=pl.DeviceIdType.MESH,
            )
        pl.semaphore_wait(barrier_sem, N_DEV - 1)

        blk_dsts = [(my + off) % N_DEV for off in SEND_OFFS] + [my]
        stages = []
        for idx, d in enumerate(blk_dsts):
            stages.append(pltpu.make_async_copy(
                x_ref.at[pl.ds(d * m_per, m_per), :],
                stage_ref.at[idx % 2],
                stage_sems.at[idx % 2],
            ))
        stages[0].start()
        stages[1].start()

        sends = []
        for idx, off in enumerate(SEND_OFFS):
            dst = (my + off) % N_DEV
            stages[idx].wait()
            blk = stage_ref[idx % 2]
            s = jnp.maximum(jnp.max(jnp.abs(blk)), 1e-30) / 32767.0
            scale_send_ref[idx] = jnp.full((8, 128), s, jnp.float32)
            srdma = pltpu.make_async_remote_copy(
                src_ref=scale_send_ref.at[idx],
                dst_ref=scale_ref.at[my],
                send_sem=scale_send_sems.at[idx],
                recv_sem=scale_recv_sems.at[my],
                device_id=(dst,),
                device_id_type=pl.DeviceIdType.MESH,
            )
            srdma.start()
            sends.append(srdma)
            xsend_ref[idx] = jnp.clip(
                jnp.round(blk / s), -32767.0, 32767.0
            ).astype(jnp.int16)
            if idx + 2 < len(blk_dsts):
                stages[idx + 2].start()
            for c in range(NCHUNK):
                rdma = pltpu.make_async_remote_copy(
                    src_ref=xsend_ref.at[idx, pl.ds(c * m_chunk, m_chunk)],
                    dst_ref=comm_ref.at[my, pl.ds(c * m_chunk, m_chunk)],
                    send_sem=data_send_sems.at[idx, c],
                    recv_sem=data_recv_sems.at[my, c],
                    device_id=(dst,),
                    device_id_type=pl.DeviceIdType.MESH,
                )
                rdma.start()
                sends.append(rdma)

        w_order = [my] + [(my + off) % N_DEV for off in CONSUME_OFFS]
        w_copies = []
        for i, d in enumerate(w_order):
            w_copies.append(pltpu.make_async_copy(
                w_ref.at[pl.ds(d * k_loc, k_loc), :],
                wbuf_ref.at[i % 2],
                w_copy_sems.at[i % 2],
            ))
        w_copies[0].start()
        w_copies[1].start()

        stages[3].wait()
        w_copies[0].wait()
        out_ref[...] = jnp.dot(
            stage_ref[1], wbuf_ref[0],
            preferred_element_type=jnp.float32,
        )

        for i, off in enumerate(CONSUME_OFFS):
            d = (my + off) % N_DEV
            sem_idx = SEND_OFFS.index(off)
            slot = (i + 1) % 2
            if i + 2 < len(w_order):
                w_copies[i + 2].start()
            w_copies[i + 1].wait()
            srecv = pltpu.make_async_remote_copy(
                src_ref=scale_ref.at[d],
                dst_ref=scale_ref.at[d],
                send_sem=scale_send_sems.at[0],
                recv_sem=scale_recv_sems.at[d],
                device_id=(my,),
                device_id_type=pl.DeviceIdType.MESH,
            )
            srecv.wait_recv()
            s_d = scale_ref[d, 0, 0]
            for c in range(NCHUNK):
                recv = pltpu.make_async_remote_copy(
                    src_ref=comm_ref.at[d, pl.ds(c * m_chunk, m_chunk)],
                    dst_ref=comm_ref.at[d, pl.ds(c * m_chunk, m_chunk)],
                    send_sem=data_send_sems.at[sem_idx, c],
                    recv_sem=data_recv_sems.at[d, c],
                    device_id=(my,),
                    device_id_type=pl.DeviceIdType.MESH,
                )
                recv.wait_recv()
                xf = comm_ref[d, pl.ds(c * m_chunk, m_chunk), :].astype(
                    jnp.float32) * s_d
                out_ref[pl.ds(c * m_chunk, m_chunk), :] += jnp.dot(
                    xf, wbuf_ref[slot],
                    preferred_element_type=jnp.float32,
                )

        for rdma in sends:
            rdma.wait_send()

        local_amax = jnp.max(jnp.abs(out_ref[...]))
        amax_ref[my] = jnp.full((8, 128), local_amax, jnp.float32)
        amax_sends = []
        for off in range(1, N_DEV):
            dst = (my + off) % N_DEV
            rdma = pltpu.make_async_remote_copy(
                src_ref=amax_ref.at[my],
                dst_ref=amax_ref.at[my],
                send_sem=amax_send_sems.at[off - 1],
                recv_sem=amax_recv_sems.at[my],
                device_id=(dst,),
                device_id_type=pl.DeviceIdType.MESH,
            )
            rdma.start()
            amax_sends.append(rdma)
        for d in range(N_DEV):
            recv = pltpu.make_async_remote_copy(
                src_ref=amax_ref.at[d],
                dst_ref=amax_ref.at[d],
                send_sem=amax_send_sems.at[0],
                recv_sem=amax_recv_sems.at[d],
                device_id=(my,),
                device_id_type=pl.DeviceIdType.MESH,
            )

            @pl.when(my != d)
            def _():
                recv.wait_recv()

        for rdma in amax_sends:
            rdma.wait_send()

        g_amax = jnp.max(amax_ref[...])
        scale = g_amax / 448.0
        q = jnp.clip(out_ref[...] / scale, -448.0, 448.0)
        q = q.astype(jnp.float8_e4m3fn).astype(jnp.float32)
        out_ref[...] = q * scale

    return pl.pallas_call(
        body,
        out_shape=jax.ShapeDtypeStruct((m_per, n), jnp.float32),
        in_specs=[
            pl.BlockSpec(memory_space=pl.ANY),
            pl.BlockSpec(memory_space=pl.ANY),
        ],
        out_specs=pl.BlockSpec(memory_space=pltpu.VMEM),
        scratch_shapes=[
            pltpu.VMEM((N_DEV, m_per, k_loc), jnp.int16),
            pltpu.VMEM((N_DEV - 1, m_per, k_loc), jnp.int16),
            pltpu.VMEM((2, m_per, k_loc), jnp.float32),
            pltpu.VMEM((2, k_loc, n), jnp.float32),
            pltpu.VMEM((N_DEV - 1, 8, 128), jnp.float32),
            pltpu.VMEM((N_DEV, 8, 128), jnp.float32),
            pltpu.VMEM((N_DEV, 8, 128), jnp.float32),
            pltpu.SemaphoreType.DMA((N_DEV - 1, NCHUNK)),
            pltpu.SemaphoreType.DMA((N_DEV, NCHUNK)),
            pltpu.SemaphoreType.DMA((N_DEV - 1,)),
            pltpu.SemaphoreType.DMA((N_DEV,)),
            pltpu.SemaphoreType.DMA((N_DEV - 1,)),
            pltpu.SemaphoreType.DMA((N_DEV,)),
            pltpu.SemaphoreType.DMA((2,)),
            pltpu.SemaphoreType.DMA((2,)),
        ],
        compiler_params=pltpu.CompilerParams(
            collective_id=0,
            vmem_limit_bytes=56 * 1024 * 1024,
        ),
    )(x, w_mat)
